# Initial kernel scaffold; baseline (speedup 1.0000x reference)
import functools

import jax
import jax.numpy as jnp
from jax import lax
from jax.experimental import pallas as pl
from jax.experimental.pallas import tpu as pltpu
from jax.experimental.pallas import tpu_sc as plsc

N = 10000
D = 128
C = 64
E = 320000

NPAD = 10240
EPAD = 327680
NW = 32
BLK = 128
BLKS_PER_W = EPAD // NW // BLK
ROWS_PER_TILE = NPAD // 16
DEGW = 16

_MESH = plsc.VectorSubcoreMesh(core_axis_name="c", subcore_axis_name="s")


def _sc_degree(dst_p):
    @functools.partial(
        pl.kernel,
        mesh=_MESH,
        out_type=jax.ShapeDtypeStruct((2 * NPAD, DEGW), jnp.float32),
        scratch_types=[
            pltpu.VMEM((BLKS_PER_W, BLK), jnp.int32),
            pltpu.VMEM((BLK, DEGW), jnp.float32),
            pltpu.VMEM((BLK, DEGW), jnp.float32),
            pltpu.VMEM_SHARED((NPAD, DEGW), jnp.float32),
        ],
    )
    def degk(dst_hbm, out_hbm, didx, obuf, zbuf, acc):
        c = lax.axis_index("c")
        s = lax.axis_index("s")
        w = c * 16 + s

        @pl.loop(0, BLK)
        def _(i):
            zbuf[pl.ds(i, 1), :] = jnp.zeros((1, DEGW), jnp.float32)
            obuf[pl.ds(i, 1), :] = jnp.ones((1, DEGW), jnp.float32)

        @pl.loop(0, ROWS_PER_TILE // BLK)
        def _(k):
            pltpu.sync_copy(zbuf, acc.at[pl.ds(s * ROWS_PER_TILE + k * BLK, BLK)])

        plsc.subcore_barrier()
        pltpu.sync_copy(dst_hbm.at[pl.ds(w * BLKS_PER_W, BLKS_PER_W)], didx)

        @pl.loop(0, BLKS_PER_W)
        def _(j):
            pltpu.sync_copy(obuf, acc.at[didx.at[j]], add=True)

        plsc.subcore_barrier()

        @pl.loop(0, ROWS_PER_TILE // BLK)
        def _(k):
            off = s * ROWS_PER_TILE + k * BLK
            pltpu.sync_copy(acc.at[pl.ds(off, BLK)],
                            out_hbm.at[pl.ds(c * NPAD + off, BLK)])

    return degk(dst_p)


def _sc_hop(t, src_p, dst_p):
    @functools.partial(
        pl.kernel,
        mesh=_MESH,
        out_type=jax.ShapeDtypeStruct((2 * NPAD, C), jnp.float32),
        scratch_types=[
            pltpu.VMEM((BLKS_PER_W, BLK), jnp.int32),
            pltpu.VMEM((BLKS_PER_W, BLK), jnp.int32),
            pltpu.VMEM((BLK, C), jnp.float32),
            pltpu.VMEM((BLK, C), jnp.float32),
            pltpu.VMEM_SHARED((NPAD, C), jnp.float32),
            pltpu.SemaphoreType.DMA,
            pltpu.SemaphoreType.DMA,
        ],
    )
    def hop(t_hbm, src_hbm, dst_hbm, out_hbm,
            sidx, didx, buf0, buf1, acc, sem0, sem1):
        c = lax.axis_index("c")
        s = lax.axis_index("s")
        w = c * 16 + s

        @pl.loop(0, BLK)
        def _(i):
            for j in range(C // 16):
                buf0[pl.ds(i, 1), pl.ds(16 * j, 16)] = jnp.zeros(
                    (1, 16), jnp.float32)

        @pl.loop(0, ROWS_PER_TILE // BLK)
        def _(k):
            pltpu.sync_copy(buf0, acc.at[pl.ds(s * ROWS_PER_TILE + k * BLK, BLK)])

        plsc.subcore_barrier()

        pltpu.sync_copy(src_hbm.at[pl.ds(w * BLKS_PER_W, BLKS_PER_W)], sidx)
        pltpu.sync_copy(dst_hbm.at[pl.ds(w * BLKS_PER_W, BLKS_PER_W)], didx)

        pltpu.async_copy(t_hbm.at[sidx.at[0]], buf0, sem0)

        @pl.loop(0, BLKS_PER_W, step=2)
        def _(j):
            pltpu.async_copy(t_hbm.at[sidx.at[j + 1]], buf1, sem1)
            pltpu.make_async_copy(t_hbm.at[sidx.at[0]], buf0, sem0).wait()
            pltpu.sync_copy(buf0, acc.at[didx.at[j]], add=True)
            nxt = jnp.minimum(j + 2, BLKS_PER_W - 1)
            pltpu.async_copy(t_hbm.at[sidx.at[nxt]], buf0, sem0)
            pltpu.make_async_copy(t_hbm.at[sidx.at[0]], buf1, sem1).wait()
            pltpu.sync_copy(buf1, acc.at[didx.at[j + 1]], add=True)

        pltpu.make_async_copy(t_hbm.at[sidx.at[0]], buf0, sem0).wait()

        plsc.subcore_barrier()

        @pl.loop(0, ROWS_PER_TILE // BLK)
        def _(k):
            off = s * ROWS_PER_TILE + k * BLK
            pltpu.sync_copy(acc.at[pl.ds(off, BLK)],
                            out_hbm.at[pl.ds(c * NPAD + off, BLK)])

    return hop(t, src_p, dst_p)


def _tc_matmul(xp, W):
    def body(x_ref, w_ref, o_ref):
        o_ref[...] = lax.dot_general(
            x_ref[...], w_ref[...], (((1,), (1,)), ((), ())),
            preferred_element_type=jnp.float32)

    return pl.pallas_call(
        body, out_shape=jax.ShapeDtypeStruct((NPAD, C), jnp.float32))(xp, W)


def _deg_cols(d_ref):
    cnt = d_ref[0:NPAD, :] + d_ref[NPAD:2 * NPAD, :]
    deg = cnt[:, 0:1] + 1.0
    rows = lax.broadcasted_iota(jnp.int32, (NPAD, 1), 0)
    return deg, rows < N


def _tc_scale_in(h0, deg2):
    def body(h_ref, d_ref, o_ref):
        deg, valid = _deg_cols(d_ref)
        dinv = jnp.where(valid, lax.rsqrt(deg), 0.0)
        o_ref[...] = h_ref[...] * dinv

    return pl.pallas_call(
        body, out_shape=jax.ShapeDtypeStruct((NPAD, C), jnp.float32))(h0, deg2)


def _tc_mid(p, u, deg2):
    def body(p_ref, u_ref, d_ref, o_ref):
        deg, valid = _deg_cols(d_ref)
        selfw = jnp.where(valid, 1.0 / deg, 0.0)
        o_ref[...] = (p_ref[0:NPAD, :] + p_ref[NPAD:2 * NPAD, :]
                      + u_ref[...]) * selfw

    return pl.pallas_call(
        body, out_shape=jax.ShapeDtypeStruct((NPAD, C), jnp.float32))(p, u, deg2)


def _tc_final(q, w1, deg2, b2):
    def body(q_ref, w_ref, d_ref, b_ref, o_ref):
        deg, valid = _deg_cols(d_ref)
        dinv = jnp.where(valid, lax.rsqrt(deg), 0.0)
        z = (q_ref[0:NPAD, :] + q_ref[NPAD:2 * NPAD, :]
             + w_ref[...]) * dinv + b_ref[...]
        m = jnp.max(z, axis=1, keepdims=True)
        lse = jnp.log(jnp.sum(jnp.exp(z - m), axis=1, keepdims=True)) + m
        o_ref[...] = (z - lse)[0:N, :]

    return pl.pallas_call(
        body, out_shape=jax.ShapeDtypeStruct((N, C), jnp.float32))(
            q, w1, deg2, b2)


def kernel(x, edge_index, W, b):
    src = edge_index[0].astype(jnp.int32)
    dst = edge_index[1].astype(jnp.int32)
    fill = jnp.full((EPAD - E,), N, jnp.int32)
    src_p = jnp.concatenate([src, fill]).reshape(EPAD // BLK, BLK)
    dst_p = jnp.concatenate([dst, fill]).reshape(EPAD // BLK, BLK)
    xp = jnp.pad(x, ((0, NPAD - N), (0, 0)))
    b2 = jnp.reshape(b, (1, C))

    h0 = _tc_matmul(xp, W)
    deg2 = _sc_degree(dst_p)
    u = _tc_scale_in(h0, deg2)
    p = _sc_hop(u, src_p, dst_p)
    w1 = _tc_mid(p, u, deg2)
    q = _sc_hop(w1, src_p, dst_p)
    return _tc_final(q, w1, deg2, b2)


# speedup vs baseline: 16.6834x; 16.6834x over previous
import functools

import jax
import jax.numpy as jnp
from jax import lax
from jax.experimental import pallas as pl
from jax.experimental.pallas import tpu as pltpu
from jax.experimental.pallas import tpu_sc as plsc

N = 10000
D = 128
C = 64
E = 320000

NPAD = 10240
EPAD = 327680
NW = 32
BLK = 128
BLKS_PER_W = EPAD // NW // BLK
ROWS_PER_TILE = NPAD // 16
DEGW = 16

_MESH = plsc.VectorSubcoreMesh(core_axis_name="c", subcore_axis_name="s")
_SC_PARAMS = pltpu.CompilerParams(use_tc_tiling_on_sc=False)


def _sc_degree(dst_p):
    @functools.partial(
        pl.kernel,
        mesh=_MESH,
        out_type=jax.ShapeDtypeStruct((2 * NPAD, DEGW), jnp.float32),
        scratch_types=[
            pltpu.VMEM((BLKS_PER_W, BLK), jnp.int32),
            pltpu.VMEM((BLK, DEGW), jnp.float32),
            pltpu.VMEM((BLK, DEGW), jnp.float32),
            pltpu.VMEM_SHARED((NPAD, DEGW), jnp.float32),
        ],
        compiler_params=_SC_PARAMS,
    )
    def degk(dst_hbm, out_hbm, didx, obuf, zbuf, acc):
        c = lax.axis_index("c")
        s = lax.axis_index("s")
        w = c * 16 + s

        @pl.loop(0, BLK)
        def _(i):
            zbuf[pl.ds(i, 1), :] = jnp.zeros((1, DEGW), jnp.float32)
            obuf[pl.ds(i, 1), :] = jnp.ones((1, DEGW), jnp.float32)

        @pl.loop(0, ROWS_PER_TILE // BLK)
        def _(k):
            pltpu.sync_copy(zbuf, acc.at[pl.ds(s * ROWS_PER_TILE + k * BLK, BLK)])

        plsc.subcore_barrier()
        pltpu.sync_copy(dst_hbm.at[pl.ds(w * BLKS_PER_W, BLKS_PER_W)], didx)

        @pl.loop(0, BLKS_PER_W)
        def _(j):
            pltpu.sync_copy(obuf, acc.at[didx.at[j]], add=True)

        plsc.subcore_barrier()

        @pl.loop(0, ROWS_PER_TILE // BLK)
        def _(k):
            off = s * ROWS_PER_TILE + k * BLK
            pltpu.sync_copy(acc.at[pl.ds(off, BLK)],
                            out_hbm.at[pl.ds(c * NPAD + off, BLK)])

    return degk(dst_p)


def _sc_hop(t, src_p, dst_p):
    @functools.partial(
        pl.kernel,
        mesh=_MESH,
        out_type=jax.ShapeDtypeStruct((2 * NPAD, C), jnp.float32),
        scratch_types=[
            pltpu.VMEM((BLKS_PER_W, BLK), jnp.int32),
            pltpu.VMEM((BLKS_PER_W, BLK), jnp.int32),
            pltpu.VMEM((BLK, C), jnp.float32),
            pltpu.VMEM((BLK, C), jnp.float32),
            pltpu.VMEM_SHARED((NPAD, C), jnp.float32),
            pltpu.SemaphoreType.DMA,
            pltpu.SemaphoreType.DMA,
        ],
        compiler_params=_SC_PARAMS,
    )
    def hop(t_hbm, src_hbm, dst_hbm, out_hbm,
            sidx, didx, buf0, buf1, acc, sem0, sem1):
        c = lax.axis_index("c")
        s = lax.axis_index("s")
        w = c * 16 + s

        @pl.loop(0, BLK)
        def _(i):
            for j in range(C // 16):
                buf0[pl.ds(i, 1), pl.ds(16 * j, 16)] = jnp.zeros(
                    (1, 16), jnp.float32)

        @pl.loop(0, ROWS_PER_TILE // BLK)
        def _(k):
            pltpu.sync_copy(buf0, acc.at[pl.ds(s * ROWS_PER_TILE + k * BLK, BLK)])

        plsc.subcore_barrier()

        pltpu.sync_copy(src_hbm.at[pl.ds(w * BLKS_PER_W, BLKS_PER_W)], sidx)
        pltpu.sync_copy(dst_hbm.at[pl.ds(w * BLKS_PER_W, BLKS_PER_W)], didx)

        pltpu.async_copy(t_hbm.at[sidx.at[0]], buf0, sem0)

        @pl.loop(0, BLKS_PER_W, step=2)
        def _(j):
            pltpu.async_copy(t_hbm.at[sidx.at[j + 1]], buf1, sem1)
            pltpu.make_async_copy(t_hbm.at[sidx.at[0]], buf0, sem0).wait()
            pltpu.sync_copy(buf0, acc.at[didx.at[j]], add=True)
            nxt = jnp.minimum(j + 2, BLKS_PER_W - 1)
            pltpu.async_copy(t_hbm.at[sidx.at[nxt]], buf0, sem0)
            pltpu.make_async_copy(t_hbm.at[sidx.at[0]], buf1, sem1).wait()
            pltpu.sync_copy(buf1, acc.at[didx.at[j + 1]], add=True)

        pltpu.make_async_copy(t_hbm.at[sidx.at[0]], buf0, sem0).wait()

        plsc.subcore_barrier()

        @pl.loop(0, ROWS_PER_TILE // BLK)
        def _(k):
            off = s * ROWS_PER_TILE + k * BLK
            pltpu.sync_copy(acc.at[pl.ds(off, BLK)],
                            out_hbm.at[pl.ds(c * NPAD + off, BLK)])

    return hop(t, src_p, dst_p)


def _tc_matmul(xp, W):
    def body(x_ref, w_ref, o_ref):
        o_ref[...] = lax.dot_general(
            x_ref[...], w_ref[...], (((1,), (1,)), ((), ())),
            preferred_element_type=jnp.float32)

    return pl.pallas_call(
        body, out_shape=jax.ShapeDtypeStruct((NPAD, C), jnp.float32))(xp, W)


def _deg_cols(d_ref):
    cnt = d_ref[0:NPAD, :] + d_ref[NPAD:2 * NPAD, :]
    deg = cnt[:, 0:1] + 1.0
    rows = lax.broadcasted_iota(jnp.int32, (NPAD, 1), 0)
    return deg, rows < N


def _tc_scale_in(h0, deg2):
    def body(h_ref, d_ref, o_ref):
        deg, valid = _deg_cols(d_ref)
        dinv = jnp.where(valid, lax.rsqrt(deg), 0.0)
        o_ref[...] = h_ref[...] * dinv

    return pl.pallas_call(
        body, out_shape=jax.ShapeDtypeStruct((NPAD, C), jnp.float32))(h0, deg2)


def _tc_mid(p, u, deg2):
    def body(p_ref, u_ref, d_ref, o_ref):
        deg, valid = _deg_cols(d_ref)
        selfw = jnp.where(valid, 1.0 / deg, 0.0)
        o_ref[...] = (p_ref[0:NPAD, :] + p_ref[NPAD:2 * NPAD, :]
                      + u_ref[...]) * selfw

    return pl.pallas_call(
        body, out_shape=jax.ShapeDtypeStruct((NPAD, C), jnp.float32))(p, u, deg2)


def _tc_final(q, w1, deg2, b2):
    def body(q_ref, w_ref, d_ref, b_ref, o_ref):
        deg, valid = _deg_cols(d_ref)
        dinv = jnp.where(valid, lax.rsqrt(deg), 0.0)
        z = (q_ref[0:NPAD, :] + q_ref[NPAD:2 * NPAD, :]
             + w_ref[...]) * dinv + b_ref[...]
        m = jnp.max(z, axis=1, keepdims=True)
        lse = jnp.log(jnp.sum(jnp.exp(z - m), axis=1, keepdims=True)) + m
        o_ref[...] = (z - lse)[0:N, :]

    return pl.pallas_call(
        body, out_shape=jax.ShapeDtypeStruct((N, C), jnp.float32))(
            q, w1, deg2, b2)


def kernel(x, edge_index, W, b):
    src = edge_index[0].astype(jnp.int32)
    dst = edge_index[1].astype(jnp.int32)
    fill = jnp.full((EPAD - E,), N, jnp.int32)
    src_p = jnp.concatenate([src, fill]).reshape(EPAD // BLK, BLK)
    dst_p = jnp.concatenate([dst, fill]).reshape(EPAD // BLK, BLK)
    xp = jnp.pad(x, ((0, NPAD - N), (0, 0)))
    b2 = jnp.reshape(b, (1, C))

    h0 = _tc_matmul(xp, W)
    deg2 = _sc_degree(dst_p)
    u = _tc_scale_in(h0, deg2)
    p = _sc_hop(u, src_p, dst_p)
    w1 = _tc_mid(p, u, deg2)
    q = _sc_hop(w1, src_p, dst_p)
    return _tc_final(q, w1, deg2, b2)


# speedup vs baseline: 38.1706x; 2.2879x over previous
import functools

import jax
import jax.numpy as jnp
from jax import lax
from jax.experimental import pallas as pl
from jax.experimental.pallas import tpu as pltpu
from jax.experimental.pallas import tpu_sc as plsc

N = 10000
D = 128
C = 64
E = 320000

NPAD = 10240
EPAD = 327680
NW = 32
BLK = 128
BLKS_PER_W = EPAD // NW // BLK
ROWS_PER_TILE = NPAD // 16
DEGW = 16

_MESH = plsc.VectorSubcoreMesh(core_axis_name="c", subcore_axis_name="s")
_SC_PARAMS = pltpu.CompilerParams(use_tc_tiling_on_sc=False)


def _sc_degree(dst_p):
    @functools.partial(
        pl.kernel,
        mesh=_MESH,
        out_type=jax.ShapeDtypeStruct((2 * NPAD, DEGW), jnp.float32),
        scratch_types=[
            pltpu.VMEM((BLKS_PER_W, BLK), jnp.int32),
            pltpu.VMEM((BLK, DEGW), jnp.float32),
            pltpu.VMEM((BLK, DEGW), jnp.float32),
            pltpu.VMEM_SHARED((NPAD, DEGW), jnp.float32),
        ],
        compiler_params=_SC_PARAMS,
    )
    def degk(dst_hbm, out_hbm, didx, obuf, zbuf, acc):
        c = lax.axis_index("c")
        s = lax.axis_index("s")
        w = c * 16 + s

        @pl.loop(0, BLK)
        def _(i):
            zbuf[pl.ds(i, 1), :] = jnp.zeros((1, DEGW), jnp.float32)
            obuf[pl.ds(i, 1), :] = jnp.ones((1, DEGW), jnp.float32)

        @pl.loop(0, ROWS_PER_TILE // BLK)
        def _(k):
            pltpu.sync_copy(zbuf, acc.at[pl.ds(s * ROWS_PER_TILE + k * BLK, BLK)])

        plsc.subcore_barrier()
        pltpu.sync_copy(dst_hbm.at[pl.ds(w * BLKS_PER_W, BLKS_PER_W)], didx)

        @pl.loop(0, BLKS_PER_W)
        def _(j):
            pltpu.sync_copy(obuf, acc.at[didx.at[j]], add=True)

        plsc.subcore_barrier()

        @pl.loop(0, ROWS_PER_TILE // BLK)
        def _(k):
            off = s * ROWS_PER_TILE + k * BLK
            pltpu.sync_copy(acc.at[pl.ds(off, BLK)],
                            out_hbm.at[pl.ds(c * NPAD + off, BLK)])

    return degk(dst_p)


def _sc_hop(t, src_p, dst_p):
    @functools.partial(
        pl.kernel,
        mesh=_MESH,
        out_type=jax.ShapeDtypeStruct((2 * NPAD, C), jnp.float32),
        scratch_types=[
            pltpu.VMEM((BLKS_PER_W, BLK), jnp.int32),
            pltpu.VMEM((BLKS_PER_W, BLK), jnp.int32),
            pltpu.VMEM((BLK, C), jnp.float32),
            pltpu.VMEM((BLK, C), jnp.float32),
            pltpu.VMEM_SHARED((NPAD, C), jnp.float32),
            pltpu.SemaphoreType.DMA,
            pltpu.SemaphoreType.DMA,
        ],
        compiler_params=_SC_PARAMS,
    )
    def hop(t_hbm, src_hbm, dst_hbm, out_hbm,
            sidx, didx, buf0, buf1, acc, sem0, sem1):
        c = lax.axis_index("c")
        s = lax.axis_index("s")
        w = c * 16 + s

        @pl.loop(0, BLK)
        def _(i):
            for j in range(C // 16):
                buf0[pl.ds(i, 1), pl.ds(16 * j, 16)] = jnp.zeros(
                    (1, 16), jnp.float32)

        @pl.loop(0, ROWS_PER_TILE // BLK)
        def _(k):
            pltpu.sync_copy(buf0, acc.at[pl.ds(s * ROWS_PER_TILE + k * BLK, BLK)])

        plsc.subcore_barrier()

        pltpu.sync_copy(src_hbm.at[pl.ds(w * BLKS_PER_W, BLKS_PER_W)], sidx)
        pltpu.sync_copy(dst_hbm.at[pl.ds(w * BLKS_PER_W, BLKS_PER_W)], didx)

        pltpu.async_copy(t_hbm.at[sidx.at[0]], buf0, sem0)

        @pl.loop(0, BLKS_PER_W, step=2)
        def _(j):
            pltpu.async_copy(t_hbm.at[sidx.at[j + 1]], buf1, sem1)
            pltpu.make_async_copy(t_hbm.at[sidx.at[0]], buf0, sem0).wait()
            pltpu.sync_copy(buf0, acc.at[didx.at[j]], add=True)
            nxt = jnp.minimum(j + 2, BLKS_PER_W - 1)
            pltpu.async_copy(t_hbm.at[sidx.at[nxt]], buf0, sem0)
            pltpu.make_async_copy(t_hbm.at[sidx.at[0]], buf1, sem1).wait()
            pltpu.sync_copy(buf1, acc.at[didx.at[j + 1]], add=True)

        pltpu.make_async_copy(t_hbm.at[sidx.at[0]], buf0, sem0).wait()

        plsc.subcore_barrier()

        @pl.loop(0, ROWS_PER_TILE // BLK)
        def _(k):
            off = s * ROWS_PER_TILE + k * BLK
            pltpu.sync_copy(acc.at[pl.ds(off, BLK)],
                            out_hbm.at[pl.ds(c * NPAD + off, BLK)])

    return hop(t, src_p, dst_p)


def _tc_matmul(xp, W):
    def body(x_ref, w_ref, o_ref):
        o_ref[...] = lax.dot_general(
            x_ref[...], w_ref[...], (((1,), (1,)), ((), ())),
            preferred_element_type=jnp.float32)

    return pl.pallas_call(
        body, out_shape=jax.ShapeDtypeStruct((NPAD, C), jnp.float32))(xp, W)


def _deg_cols(d_ref):
    cnt = d_ref[0:NPAD, :] + d_ref[NPAD:2 * NPAD, :]
    deg = cnt[:, 0:1] + 1.0
    rows = lax.broadcasted_iota(jnp.int32, (NPAD, 1), 0)
    return deg, rows < N


def _tc_scale_in(h0, deg2):
    def body(h_ref, d_ref, o_ref):
        deg, valid = _deg_cols(d_ref)
        dinv = jnp.where(valid, lax.rsqrt(deg), 0.0)
        o_ref[...] = h_ref[...] * dinv

    return pl.pallas_call(
        body, out_shape=jax.ShapeDtypeStruct((NPAD, C), jnp.float32))(h0, deg2)


def _tc_mid(p, u, deg2):
    def body(p_ref, u_ref, d_ref, o_ref):
        deg, valid = _deg_cols(d_ref)
        selfw = jnp.where(valid, 1.0 / deg, 0.0)
        o_ref[...] = (p_ref[0:NPAD, :] + p_ref[NPAD:2 * NPAD, :]
                      + u_ref[...]) * selfw

    return pl.pallas_call(
        body, out_shape=jax.ShapeDtypeStruct((NPAD, C), jnp.float32))(p, u, deg2)


def _tc_final(q, w1, deg2, b2):
    def body(q_ref, w_ref, d_ref, b_ref, o_ref):
        deg, valid = _deg_cols(d_ref)
        dinv = jnp.where(valid, lax.rsqrt(deg), 0.0)
        z = (q_ref[0:NPAD, :] + q_ref[NPAD:2 * NPAD, :]
             + w_ref[...]) * dinv + b_ref[...]
        m = jnp.max(z, axis=1, keepdims=True)
        lse = jnp.log(jnp.sum(jnp.exp(z - m), axis=1, keepdims=True)) + m
        o_ref[...] = (z - lse)[0:N, :]

    return pl.pallas_call(
        body, out_shape=jax.ShapeDtypeStruct((N, C), jnp.float32))(
            q, w1, deg2, b2)


def kernel(x, edge_index, W, b):
    src = edge_index[0].astype(jnp.int32)
    dst = edge_index[1].astype(jnp.int32)
    fill = N + jnp.arange(EPAD - E, dtype=jnp.int32) % (NPAD - N)
    src_p = jnp.concatenate([src, fill]).reshape(EPAD // BLK, BLK)
    dst_p = jnp.concatenate([dst, fill]).reshape(EPAD // BLK, BLK)
    xp = jnp.pad(x, ((0, NPAD - N), (0, 0)))
    b2 = jnp.reshape(b, (1, C))

    h0 = _tc_matmul(xp, W)
    deg2 = _sc_degree(dst_p)
    u = _tc_scale_in(h0, deg2)
    p = _sc_hop(u, src_p, dst_p)
    w1 = _tc_mid(p, u, deg2)
    q = _sc_hop(w1, src_p, dst_p)
    return _tc_final(q, w1, deg2, b2)


# speedup vs baseline: 39.0628x; 1.0234x over previous
import functools

import jax
import jax.numpy as jnp
from jax import lax
from jax.experimental import pallas as pl
from jax.experimental.pallas import tpu as pltpu
from jax.experimental.pallas import tpu_sc as plsc

N = 10000
D = 128
C = 64
E = 320000

NPAD = 10240
EPAD = 327680
NW = 32
BLK = 128
BLKS_PER_W = EPAD // NW // BLK
ROWS_PER_TILE = NPAD // 16
DEGW = 16

_MESH = plsc.VectorSubcoreMesh(core_axis_name="c", subcore_axis_name="s")
_SC_PARAMS = pltpu.CompilerParams(use_tc_tiling_on_sc=False)


def _sc_degree(dst_p):
    @functools.partial(
        pl.kernel,
        mesh=_MESH,
        out_type=jax.ShapeDtypeStruct((2 * NPAD, DEGW), jnp.float32),
        scratch_types=[
            pltpu.VMEM((BLKS_PER_W, BLK), jnp.int32),
            pltpu.VMEM((BLK, DEGW), jnp.float32),
            pltpu.VMEM((BLK, DEGW), jnp.float32),
            pltpu.VMEM_SHARED((NPAD, DEGW), jnp.float32),
        ],
        compiler_params=_SC_PARAMS,
    )
    def degk(dst_hbm, out_hbm, didx, obuf, zbuf, acc):
        c = lax.axis_index("c")
        s = lax.axis_index("s")
        w = c * 16 + s

        @pl.loop(0, BLK)
        def _(i):
            zbuf[pl.ds(i, 1), :] = jnp.zeros((1, DEGW), jnp.float32)
            obuf[pl.ds(i, 1), :] = jnp.ones((1, DEGW), jnp.float32)

        @pl.loop(0, ROWS_PER_TILE // BLK)
        def _(k):
            pltpu.sync_copy(zbuf, acc.at[pl.ds(s * ROWS_PER_TILE + k * BLK, BLK)])

        plsc.subcore_barrier()
        pltpu.sync_copy(dst_hbm.at[pl.ds(w * BLKS_PER_W, BLKS_PER_W)], didx)

        @pl.loop(0, BLKS_PER_W)
        def _(j):
            pltpu.sync_copy(obuf, acc.at[didx.at[j]], add=True)

        plsc.subcore_barrier()

        @pl.loop(0, ROWS_PER_TILE // BLK)
        def _(k):
            off = s * ROWS_PER_TILE + k * BLK
            pltpu.sync_copy(acc.at[pl.ds(off, BLK)],
                            out_hbm.at[pl.ds(c * NPAD + off, BLK)])

    return degk(dst_p)


def _sc_hop(t, src_p, dst_p):
    @functools.partial(
        pl.kernel,
        mesh=_MESH,
        out_type=jax.ShapeDtypeStruct((2 * NPAD, C), jnp.float32),
        scratch_types=[
            pltpu.VMEM((BLKS_PER_W, BLK), jnp.int32),
            pltpu.VMEM((BLKS_PER_W, BLK), jnp.int32),
            pltpu.VMEM((4, BLK, C), jnp.float32),
            pltpu.VMEM_SHARED((NPAD, C), jnp.float32),
            pltpu.SemaphoreType.DMA,
            pltpu.SemaphoreType.DMA,
            pltpu.SemaphoreType.DMA,
            pltpu.SemaphoreType.DMA,
            pltpu.SemaphoreType.DMA,
            pltpu.SemaphoreType.DMA,
            pltpu.SemaphoreType.DMA,
            pltpu.SemaphoreType.DMA,
        ],
        compiler_params=_SC_PARAMS,
    )
    def hop(t_hbm, src_hbm, dst_hbm, out_hbm,
            sidx, didx, bufs, acc, g0, g1, g2, g3, s0, s1, s2, s3):
        c = lax.axis_index("c")
        s = lax.axis_index("s")
        w = c * 16 + s
        gsem = (g0, g1, g2, g3)
        ssem = (s0, s1, s2, s3)

        def gather(blk, b, sem):
            pltpu.async_copy(t_hbm.at[sidx.at[blk]], bufs.at[b], sem)

        def gwait(b):
            pltpu.make_async_copy(t_hbm.at[sidx.at[0]], bufs.at[b],
                                  gsem[b]).wait()

        def scat(blk, b):
            pltpu.async_copy(bufs.at[b], acc.at[didx.at[blk]], ssem[b],
                             add=True)

        def swait(b):
            pltpu.make_async_copy(bufs.at[b], acc.at[didx.at[0]],
                                  ssem[b]).wait()

        @pl.loop(0, BLK)
        def _(i):
            for j in range(C // 16):
                bufs[pl.ds(0, 1), pl.ds(i, 1), pl.ds(16 * j, 16)] = jnp.zeros(
                    (1, 1, 16), jnp.float32)

        @pl.loop(0, ROWS_PER_TILE // BLK)
        def _(k):
            pltpu.sync_copy(bufs.at[0],
                            acc.at[pl.ds(s * ROWS_PER_TILE + k * BLK, BLK)])

        plsc.subcore_barrier()

        pltpu.sync_copy(src_hbm.at[pl.ds(w * BLKS_PER_W, BLKS_PER_W)], sidx)
        pltpu.sync_copy(dst_hbm.at[pl.ds(w * BLKS_PER_W, BLKS_PER_W)], didx)

        gather(0, 0, g0)
        gather(1, 1, g1)
        gwait(0); scat(0, 0); gather(2, 2, g2)
        gwait(1); scat(1, 1); gather(3, 3, g3)
        gwait(2); scat(2, 2); swait(0); gather(4, 0, g0)
        gwait(3); scat(3, 3); swait(1); gather(5, 1, g1)

        @pl.loop(4, BLKS_PER_W, step=4)
        def _(j):
            for b in range(4):
                blk = j + b
                gwait(b)
                scat(blk, b)
                nb = (b + 2) % 4
                swait(nb)
                nxt = jnp.minimum(blk + 2, BLKS_PER_W - 1)
                gather(nxt, nb, gsem[nb])

        gwait(0)
        gwait(1)
        swait(2)
        swait(3)

        plsc.subcore_barrier()

        off = s * ROWS_PER_TILE
        pltpu.sync_copy(acc.at[pl.ds(off, ROWS_PER_TILE)],
                        out_hbm.at[pl.ds(c * NPAD + off, ROWS_PER_TILE)])

    return hop(t, src_p, dst_p)


def _tc_matmul(xp, W):
    def body(x_ref, w_ref, o_ref):
        o_ref[...] = lax.dot_general(
            x_ref[...], w_ref[...], (((1,), (1,)), ((), ())),
            preferred_element_type=jnp.float32)

    return pl.pallas_call(
        body, out_shape=jax.ShapeDtypeStruct((NPAD, C), jnp.float32))(xp, W)


def _deg_cols(d_ref):
    cnt = d_ref[0:NPAD, :] + d_ref[NPAD:2 * NPAD, :]
    deg = cnt[:, 0:1] + 1.0
    rows = lax.broadcasted_iota(jnp.int32, (NPAD, 1), 0)
    return deg, rows < N


def _tc_scale_in(h0, deg2):
    def body(h_ref, d_ref, o_ref):
        deg, valid = _deg_cols(d_ref)
        dinv = jnp.where(valid, lax.rsqrt(deg), 0.0)
        o_ref[...] = h_ref[...] * dinv

    return pl.pallas_call(
        body, out_shape=jax.ShapeDtypeStruct((NPAD, C), jnp.float32))(h0, deg2)


def _tc_mid(p, u, deg2):
    def body(p_ref, u_ref, d_ref, o_ref):
        deg, valid = _deg_cols(d_ref)
        selfw = jnp.where(valid, 1.0 / deg, 0.0)
        o_ref[...] = (p_ref[0:NPAD, :] + p_ref[NPAD:2 * NPAD, :]
                      + u_ref[...]) * selfw

    return pl.pallas_call(
        body, out_shape=jax.ShapeDtypeStruct((NPAD, C), jnp.float32))(p, u, deg2)


def _tc_final(q, w1, deg2, b2):
    def body(q_ref, w_ref, d_ref, b_ref, o_ref):
        deg, valid = _deg_cols(d_ref)
        dinv = jnp.where(valid, lax.rsqrt(deg), 0.0)
        z = (q_ref[0:NPAD, :] + q_ref[NPAD:2 * NPAD, :]
             + w_ref[...]) * dinv + b_ref[...]
        m = jnp.max(z, axis=1, keepdims=True)
        lse = jnp.log(jnp.sum(jnp.exp(z - m), axis=1, keepdims=True)) + m
        o_ref[...] = (z - lse)[0:N, :]

    return pl.pallas_call(
        body, out_shape=jax.ShapeDtypeStruct((N, C), jnp.float32))(
            q, w1, deg2, b2)


def kernel(x, edge_index, W, b):
    src = edge_index[0].astype(jnp.int32)
    dst = edge_index[1].astype(jnp.int32)
    fill = N + jnp.arange(EPAD - E, dtype=jnp.int32) % (NPAD - N)
    src_p = jnp.concatenate([src, fill]).reshape(EPAD // BLK, BLK)
    dst_p = jnp.concatenate([dst, fill]).reshape(EPAD // BLK, BLK)
    xp = jnp.pad(x, ((0, NPAD - N), (0, 0)))
    b2 = jnp.reshape(b, (1, C))

    h0 = _tc_matmul(xp, W)
    deg2 = _sc_degree(dst_p)
    u = _tc_scale_in(h0, deg2)
    p = _sc_hop(u, src_p, dst_p)
    w1 = _tc_mid(p, u, deg2)
    q = _sc_hop(w1, src_p, dst_p)
    return _tc_final(q, w1, deg2, b2)


# speedup vs baseline: 40.4649x; 1.0359x over previous
import functools

import jax
import jax.numpy as jnp
from jax import lax
from jax.experimental import pallas as pl
from jax.experimental.pallas import tpu as pltpu
from jax.experimental.pallas import tpu_sc as plsc

N = 10000
D = 128
C = 64
E = 320000

NPAD = 10240
EPAD = 327680
NW = 32
BLK = 128
BLKS_PER_W = EPAD // NW // BLK
ROWS_PER_TILE = NPAD // 16
DEGW = 16

_MESH = plsc.VectorSubcoreMesh(core_axis_name="c", subcore_axis_name="s")
_SC_PARAMS = pltpu.CompilerParams(use_tc_tiling_on_sc=False)


def _sc_degree(dst_p):
    @functools.partial(
        pl.kernel,
        mesh=_MESH,
        out_type=jax.ShapeDtypeStruct((2 * NPAD, DEGW), jnp.float32),
        scratch_types=[
            pltpu.VMEM((BLKS_PER_W, BLK), jnp.int32),
            pltpu.VMEM((BLK, DEGW), jnp.float32),
            pltpu.VMEM((BLK, DEGW), jnp.float32),
            pltpu.VMEM_SHARED((NPAD, DEGW), jnp.float32),
            pltpu.SemaphoreType.DMA,
            pltpu.SemaphoreType.DMA,
        ],
        compiler_params=_SC_PARAMS,
    )
    def degk(dst_hbm, out_hbm, didx, obuf, zbuf, acc, isem, zsem):
        c = lax.axis_index("c")
        s = lax.axis_index("s")
        w = c * 16 + s

        ih = pltpu.async_copy(dst_hbm.at[pl.ds(w * BLKS_PER_W, BLKS_PER_W)],
                              didx, isem)

        @pl.loop(0, BLK)
        def _(i):
            zbuf[pl.ds(i, 1), :] = jnp.zeros((1, DEGW), jnp.float32)
            obuf[pl.ds(i, 1), :] = jnp.ones((1, DEGW), jnp.float32)

        for k in range(ROWS_PER_TILE // BLK):
            pltpu.async_copy(
                zbuf, acc.at[pl.ds(s * ROWS_PER_TILE + k * BLK, BLK)], zsem)
        for k in range(ROWS_PER_TILE // BLK):
            pltpu.make_async_copy(
                zbuf, acc.at[pl.ds(s * ROWS_PER_TILE + k * BLK, BLK)],
                zsem).wait()
        ih.wait()
        plsc.subcore_barrier()

        @pl.loop(0, BLKS_PER_W)
        def _(j):
            pltpu.sync_copy(obuf, acc.at[didx.at[j]], add=True)

        plsc.subcore_barrier()

        off = s * ROWS_PER_TILE
        pltpu.sync_copy(acc.at[pl.ds(off, ROWS_PER_TILE)],
                        out_hbm.at[pl.ds(c * NPAD + off, ROWS_PER_TILE)])

    return degk(dst_p)


def _sc_hop(t, src_p, dst_p):
    @functools.partial(
        pl.kernel,
        mesh=_MESH,
        out_type=jax.ShapeDtypeStruct((2 * NPAD, C), jnp.float32),
        scratch_types=[
            pltpu.VMEM((BLKS_PER_W, BLK), jnp.int32),
            pltpu.VMEM((BLKS_PER_W, BLK), jnp.int32),
            pltpu.VMEM((4, BLK, C), jnp.float32),
            pltpu.VMEM_SHARED((NPAD, C), jnp.float32),
            pltpu.SemaphoreType.DMA,
            pltpu.SemaphoreType.DMA,
            pltpu.SemaphoreType.DMA,
            pltpu.SemaphoreType.DMA,
            pltpu.SemaphoreType.DMA,
            pltpu.SemaphoreType.DMA,
            pltpu.SemaphoreType.DMA,
            pltpu.SemaphoreType.DMA,
        ],
        compiler_params=_SC_PARAMS,
    )
    def hop(t_hbm, src_hbm, dst_hbm, out_hbm,
            sidx, didx, bufs, acc, g0, g1, g2, g3, s0, s1, s2, s3):
        c = lax.axis_index("c")
        s = lax.axis_index("s")
        w = c * 16 + s
        gsem = (g0, g1, g2, g3)
        ssem = (s0, s1, s2, s3)

        def gather(blk, b, sem):
            pltpu.async_copy(t_hbm.at[sidx.at[blk]], bufs.at[b], sem)

        def gwait(b):
            pltpu.make_async_copy(t_hbm.at[sidx.at[0]], bufs.at[b],
                                  gsem[b]).wait()

        def scat(blk, b):
            pltpu.async_copy(bufs.at[b], acc.at[didx.at[blk]], ssem[b],
                             add=True)

        def swait(b):
            pltpu.make_async_copy(bufs.at[b], acc.at[didx.at[0]],
                                  ssem[b]).wait()

        ih1 = pltpu.async_copy(
            src_hbm.at[pl.ds(w * BLKS_PER_W, BLKS_PER_W)], sidx, g2)
        ih2 = pltpu.async_copy(
            dst_hbm.at[pl.ds(w * BLKS_PER_W, BLKS_PER_W)], didx, g3)

        @pl.loop(0, BLK)
        def _(i):
            for j in range(C // 16):
                bufs[pl.ds(0, 1), pl.ds(i, 1), pl.ds(16 * j, 16)] = jnp.zeros(
                    (1, 1, 16), jnp.float32)

        for k in range(ROWS_PER_TILE // BLK):
            pltpu.async_copy(
                bufs.at[0], acc.at[pl.ds(s * ROWS_PER_TILE + k * BLK, BLK)],
                s0)
        for k in range(ROWS_PER_TILE // BLK):
            pltpu.make_async_copy(
                bufs.at[0], acc.at[pl.ds(s * ROWS_PER_TILE + k * BLK, BLK)],
                s0).wait()
        ih1.wait()
        ih2.wait()

        gather(0, 0, g0)
        gather(1, 1, g1)
        plsc.subcore_barrier()

        gwait(0); scat(0, 0); gather(2, 2, g2)
        gwait(1); scat(1, 1); gather(3, 3, g3)
        gwait(2); scat(2, 2); swait(0); gather(4, 0, g0)
        gwait(3); scat(3, 3); swait(1); gather(5, 1, g1)

        @pl.loop(4, BLKS_PER_W, step=4)
        def _(j):
            for b in range(4):
                blk = j + b
                gwait(b)
                scat(blk, b)
                nb = (b + 2) % 4
                swait(nb)
                nxt = jnp.minimum(blk + 2, BLKS_PER_W - 1)
                gather(nxt, nb, gsem[nb])

        gwait(0)
        gwait(1)
        swait(2)
        swait(3)

        plsc.subcore_barrier()

        off = s * ROWS_PER_TILE
        pltpu.sync_copy(acc.at[pl.ds(off, ROWS_PER_TILE)],
                        out_hbm.at[pl.ds(c * NPAD + off, ROWS_PER_TILE)])

    return hop(t, src_p, dst_p)


def _deg_cols(d_ref):
    cnt = d_ref[0:NPAD, :] + d_ref[NPAD:2 * NPAD, :]
    deg = cnt[:, 0:1] + 1.0
    rows = lax.broadcasted_iota(jnp.int32, (NPAD, 1), 0)
    return deg, rows < N


def _tc_input(xp, W, deg2):
    def body(x_ref, w_ref, d_ref, o_ref):
        deg, valid = _deg_cols(d_ref)
        dinv = jnp.where(valid, lax.rsqrt(deg), 0.0)
        h0 = lax.dot_general(
            x_ref[...], w_ref[...], (((1,), (1,)), ((), ())),
            preferred_element_type=jnp.float32)
        o_ref[...] = h0 * dinv

    return pl.pallas_call(
        body, out_shape=jax.ShapeDtypeStruct((NPAD, C), jnp.float32))(
            xp, W, deg2)


def _tc_mid(p, u, deg2):
    def body(p_ref, u_ref, d_ref, o_ref):
        deg, valid = _deg_cols(d_ref)
        selfw = jnp.where(valid, 1.0 / deg, 0.0)
        o_ref[...] = (p_ref[0:NPAD, :] + p_ref[NPAD:2 * NPAD, :]
                      + u_ref[...]) * selfw

    return pl.pallas_call(
        body, out_shape=jax.ShapeDtypeStruct((NPAD, C), jnp.float32))(p, u, deg2)


def _tc_final(q, w1, deg2, b2):
    def body(q_ref, w_ref, d_ref, b_ref, o_ref):
        deg, valid = _deg_cols(d_ref)
        dinv = jnp.where(valid, lax.rsqrt(deg), 0.0)
        z = (q_ref[0:NPAD, :] + q_ref[NPAD:2 * NPAD, :]
             + w_ref[...]) * dinv + b_ref[...]
        m = jnp.max(z, axis=1, keepdims=True)
        lse = jnp.log(jnp.sum(jnp.exp(z - m), axis=1, keepdims=True)) + m
        o_ref[...] = (z - lse)[0:N, :]

    return pl.pallas_call(
        body, out_shape=jax.ShapeDtypeStruct((N, C), jnp.float32))(
            q, w1, deg2, b2)


def kernel(x, edge_index, W, b):
    src = edge_index[0].astype(jnp.int32)
    dst = edge_index[1].astype(jnp.int32)
    fill = N + jnp.arange(EPAD - E, dtype=jnp.int32) % (NPAD - N)
    src_p = jnp.concatenate([src, fill]).reshape(EPAD // BLK, BLK)
    dst_p = jnp.concatenate([dst, fill]).reshape(EPAD // BLK, BLK)
    xp = jnp.pad(x, ((0, NPAD - N), (0, 0)))
    b2 = jnp.reshape(b, (1, C))

    deg2 = _sc_degree(dst_p)
    u = _tc_input(xp, W, deg2)
    p = _sc_hop(u, src_p, dst_p)
    w1 = _tc_mid(p, u, deg2)
    q = _sc_hop(w1, src_p, dst_p)
    return _tc_final(q, w1, deg2, b2)


# speedup vs baseline: 43.8188x; 1.0829x over previous
import functools

import jax
import jax.numpy as jnp
from jax import lax
from jax.experimental import pallas as pl
from jax.experimental.pallas import tpu as pltpu
from jax.experimental.pallas import tpu_sc as plsc

N = 10000
D = 128
C = 64
E = 320000

NPAD = 10240
EPAD = 327680
NW = 32
BLK = 128
BLKS_PER_W = EPAD // NW // BLK
ROWS_PER_TILE = NPAD // 16
DEGW = 16

_MESH = plsc.VectorSubcoreMesh(core_axis_name="c", subcore_axis_name="s")
_SC_PARAMS = pltpu.CompilerParams(use_tc_tiling_on_sc=False)


def _sc_degree(dst_p):
    @functools.partial(
        pl.kernel,
        mesh=_MESH,
        out_type=jax.ShapeDtypeStruct((2 * NPAD, DEGW), jnp.float32),
        scratch_types=[
            pltpu.VMEM((BLKS_PER_W, BLK), jnp.int32),
            pltpu.VMEM((BLK, DEGW), jnp.float32),
            pltpu.VMEM((BLK, DEGW), jnp.float32),
            pltpu.VMEM_SHARED((NPAD, DEGW), jnp.float32),
            pltpu.SemaphoreType.DMA,
            pltpu.SemaphoreType.DMA,
        ],
        compiler_params=_SC_PARAMS,
    )
    def degk(dst_hbm, out_hbm, didx, obuf, zbuf, acc, isem, zsem):
        c = lax.axis_index("c")
        s = lax.axis_index("s")
        w = c * 16 + s

        ih = pltpu.async_copy(dst_hbm.at[pl.ds(w * BLKS_PER_W, BLKS_PER_W)],
                              didx, isem)

        @pl.loop(0, BLK)
        def _(i):
            zbuf[pl.ds(i, 1), :] = jnp.zeros((1, DEGW), jnp.float32)
            obuf[pl.ds(i, 1), :] = jnp.ones((1, DEGW), jnp.float32)

        for k in range(ROWS_PER_TILE // BLK):
            pltpu.async_copy(
                zbuf, acc.at[pl.ds(s * ROWS_PER_TILE + k * BLK, BLK)], zsem)
        for k in range(ROWS_PER_TILE // BLK):
            pltpu.make_async_copy(
                zbuf, acc.at[pl.ds(s * ROWS_PER_TILE + k * BLK, BLK)],
                zsem).wait()
        ih.wait()
        plsc.subcore_barrier()

        @pl.loop(0, BLKS_PER_W)
        def _(j):
            pltpu.sync_copy(obuf, acc.at[didx.at[j]], add=True)

        plsc.subcore_barrier()

        off = s * ROWS_PER_TILE
        pltpu.sync_copy(acc.at[pl.ds(off, ROWS_PER_TILE)],
                        out_hbm.at[pl.ds(c * NPAD + off, ROWS_PER_TILE)])

    return degk(dst_p)


def _sc_hop(t, src_p, dst_p):
    @functools.partial(
        pl.kernel,
        mesh=_MESH,
        out_type=jax.ShapeDtypeStruct((2 * NPAD, C), jnp.float32),
        scratch_types=[
            pltpu.VMEM((BLKS_PER_W, BLK), jnp.int32),
            pltpu.VMEM((BLKS_PER_W, BLK), jnp.int32),
            pltpu.VMEM((4, BLK, C), jnp.float32),
            pltpu.VMEM_SHARED((NPAD, C), jnp.float32),
            pltpu.SemaphoreType.DMA,
            pltpu.SemaphoreType.DMA,
            pltpu.SemaphoreType.DMA,
            pltpu.SemaphoreType.DMA,
            pltpu.SemaphoreType.DMA,
            pltpu.SemaphoreType.DMA,
            pltpu.SemaphoreType.DMA,
            pltpu.SemaphoreType.DMA,
        ],
        compiler_params=_SC_PARAMS,
    )
    def hop(t_hbm, src_hbm, dst_hbm, out_hbm,
            sidx, didx, bufs, acc, g0, g1, g2, g3, s0, s1, s2, s3):
        c = lax.axis_index("c")
        s = lax.axis_index("s")
        w = c * 16 + s
        gsem = (g0, g1, g2, g3)
        ssem = (s0, s1, s2, s3)

        def gather(blk, b, sem):
            pltpu.async_copy(t_hbm.at[sidx.at[blk]], bufs.at[b], sem)

        def gwait(b):
            pltpu.make_async_copy(t_hbm.at[sidx.at[0]], bufs.at[b],
                                  gsem[b]).wait()

        def scat(blk, b):
            pltpu.async_copy(bufs.at[b], acc.at[didx.at[blk]], ssem[b],
                             add=True)

        def swait(b):
            pltpu.make_async_copy(bufs.at[b], acc.at[didx.at[0]],
                                  ssem[b]).wait()

        ih1 = pltpu.async_copy(
            src_hbm.at[pl.ds(w * BLKS_PER_W, BLKS_PER_W)], sidx, g2)
        ih2 = pltpu.async_copy(
            dst_hbm.at[pl.ds(w * BLKS_PER_W, BLKS_PER_W)], didx, g3)

        @pl.loop(0, BLK)
        def _(i):
            for j in range(C // 16):
                bufs[pl.ds(0, 1), pl.ds(i, 1), pl.ds(16 * j, 16)] = jnp.zeros(
                    (1, 1, 16), jnp.float32)

        for k in range(ROWS_PER_TILE // BLK):
            pltpu.async_copy(
                bufs.at[0], acc.at[pl.ds(s * ROWS_PER_TILE + k * BLK, BLK)],
                s0)
        for k in range(ROWS_PER_TILE // BLK):
            pltpu.make_async_copy(
                bufs.at[0], acc.at[pl.ds(s * ROWS_PER_TILE + k * BLK, BLK)],
                s0).wait()
        ih1.wait()
        ih2.wait()

        gather(0, 0, g0)
        gather(1, 1, g1)
        plsc.subcore_barrier()

        gwait(0); scat(0, 0); gather(2, 2, g2)
        gwait(1); scat(1, 1); gather(3, 3, g3)
        gwait(2); scat(2, 2); swait(0); gather(4, 0, g0)
        gwait(3); scat(3, 3); swait(1); gather(5, 1, g1)

        @pl.loop(4, BLKS_PER_W, step=4)
        def _(j):
            for b in range(4):
                blk = j + b
                gwait(b)
                scat(blk, b)
                nb = (b + 2) % 4
                swait(nb)
                nxt = jnp.minimum(blk + 2, BLKS_PER_W - 1)
                gather(nxt, nb, gsem[nb])

        gwait(0)
        gwait(1)
        swait(2)
        swait(3)

        plsc.subcore_barrier()

        off = s * ROWS_PER_TILE
        pltpu.sync_copy(acc.at[pl.ds(off, ROWS_PER_TILE)],
                        out_hbm.at[pl.ds(c * NPAD + off, ROWS_PER_TILE)])

    return hop(t, src_p, dst_p)


NH = NPAD // 2


def _dinv_bcast(d_ref):
    cnt = d_ref[0:NPAD, :] + d_ref[NPAD:2 * NPAD, :]
    cnt3 = cnt.reshape(NH, 2, DEGW)
    cnt32 = jnp.concatenate([cnt3[:, 0, :], cnt3[:, 1, :]], axis=1)
    deg32 = cnt32 + 1.0
    valid = lax.broadcasted_iota(jnp.int32, (NH, 1), 0) < N // 2
    dinv32 = jnp.where(valid, lax.rsqrt(deg32), 0.0)
    kk = lax.broadcasted_iota(jnp.int32, (32, 128), 0)
    jj = lax.broadcasted_iota(jnp.int32, (32, 128), 1)
    g = jnp.where((kk < 16) == (jj < C), 1.0 / 16.0, 0.0)
    return lax.dot_general(dinv32, g, (((1,), (0,)), ((), ())),
                           preferred_element_type=jnp.float32)


def _tc_input(x3, W, degv):
    def body(x_ref, w_ref, d_ref, o_ref):
        s = _dinv_bcast(d_ref)
        dn = (((1,), (1,)), ((), ()))
        he = lax.dot_general(x_ref[:, 0, :], w_ref[...], dn,
                             preferred_element_type=jnp.float32)
        ho = lax.dot_general(x_ref[:, 1, :], w_ref[...], dn,
                             preferred_element_type=jnp.float32)
        o_ref[...] = jnp.concatenate([he, ho], axis=1) * s

    return pl.pallas_call(
        body, out_shape=jax.ShapeDtypeStruct((NH, 128), jnp.float32))(
            x3, W, degv)


def _tc_mid(pv, u2, degv):
    def body(p_ref, u_ref, d_ref, o_ref):
        s = _dinv_bcast(d_ref)
        cmb = p_ref[0:NH, :] + p_ref[NH:2 * NH, :] + u_ref[...]
        o_ref[...] = cmb * (s * s)

    return pl.pallas_call(
        body, out_shape=jax.ShapeDtypeStruct((NH, 128), jnp.float32))(
            pv, u2, degv)


def _tc_final(qv, w2, degv, bb):
    def body(q_ref, w_ref, d_ref, b_ref, o_ref):
        s = _dinv_bcast(d_ref)
        z = (q_ref[0:NH, :] + q_ref[NH:2 * NH, :] + w_ref[...]) * s + b_ref[...]
        ze = z[:, 0:C]
        zo = z[:, C:128]
        lse_e = jnp.max(ze, axis=1, keepdims=True)
        lse_o = jnp.max(zo, axis=1, keepdims=True)
        lse_e = lse_e + jnp.log(
            jnp.sum(jnp.exp(ze - lse_e), axis=1, keepdims=True))
        lse_o = lse_o + jnp.log(
            jnp.sum(jnp.exp(zo - lse_o), axis=1, keepdims=True))
        o_ref[...] = jnp.concatenate([ze - lse_e, zo - lse_o], axis=1)

    return pl.pallas_call(
        body, out_shape=jax.ShapeDtypeStruct((NH, 128), jnp.float32))(
            qv, w2, degv, bb)


def kernel(x, edge_index, W, b):
    src = edge_index[0].astype(jnp.int32)
    dst = edge_index[1].astype(jnp.int32)
    fill = N + jnp.arange(EPAD - E, dtype=jnp.int32) % (NPAD - N)
    src_p = jnp.concatenate([src, fill]).reshape(EPAD // BLK, BLK)
    dst_p = jnp.concatenate([dst, fill]).reshape(EPAD // BLK, BLK)
    x3 = jnp.pad(x, ((0, NPAD - N), (0, 0))).reshape(NH, 2, D)
    bb = jnp.reshape(jnp.concatenate([b, b]), (1, 128))

    degv = _sc_degree(dst_p)
    u2 = _tc_input(x3, W, degv)
    p = _sc_hop(jnp.reshape(u2, (NPAD, C)), src_p, dst_p)
    w2 = _tc_mid(jnp.reshape(p, (NPAD, 128)), u2, degv)
    q = _sc_hop(jnp.reshape(w2, (NPAD, C)), src_p, dst_p)
    out2 = _tc_final(jnp.reshape(q, (NPAD, 128)), w2, degv, bb)
    return jnp.reshape(out2, (NPAD, C))[0:N]


# speedup vs baseline: 44.8272x; 1.0230x over previous
import functools

import jax
import jax.numpy as jnp
from jax import lax
from jax.experimental import pallas as pl
from jax.experimental.pallas import tpu as pltpu
from jax.experimental.pallas import tpu_sc as plsc

N = 10000
D = 128
C = 64
E = 320000

NPAD = 10240
NH = NPAD // 2
BLK = 128
EROWS = E // BLK
NW = 32
WROWS = EROWS // NW
ROWS_PER_TILE = NPAD // 16
DEGW = 16
TCB = 640

_MESH = plsc.VectorSubcoreMesh(core_axis_name="c", subcore_axis_name="s")
_SC_PARAMS = pltpu.CompilerParams(use_tc_tiling_on_sc=False)


def _sc_degree(dst_r):
    @functools.partial(
        pl.kernel,
        mesh=_MESH,
        out_type=jax.ShapeDtypeStruct((2 * NPAD, 64), jnp.float32),
        scratch_types=[
            pltpu.VMEM((WROWS + 1, BLK), jnp.int32),
            pltpu.VMEM((BLK, DEGW), jnp.float32),
            pltpu.VMEM((BLK, DEGW), jnp.float32),
            pltpu.VMEM_SHARED((NPAD, DEGW), jnp.float32),
            pltpu.SemaphoreType.DMA,
            pltpu.SemaphoreType.DMA,
        ],
        compiler_params=_SC_PARAMS,
    )
    def degk(dst_hbm, out_hbm, didx, obuf, zbuf, acc, isem, zsem):
        c = lax.axis_index("c")
        s = lax.axis_index("s")
        w = c * 16 + s

        ih = pltpu.async_copy(dst_hbm.at[pl.ds(w * WROWS, WROWS)],
                              didx.at[pl.ds(0, WROWS)], isem)

        @pl.loop(0, BLK)
        def _(i):
            zbuf[pl.ds(i, 1), :] = jnp.zeros((1, DEGW), jnp.float32)
            obuf[pl.ds(i, 1), :] = jnp.ones((1, DEGW), jnp.float32)

        for k in range(ROWS_PER_TILE // BLK):
            pltpu.async_copy(
                zbuf, acc.at[pl.ds(s * ROWS_PER_TILE + k * BLK, BLK)], zsem)
        for k in range(ROWS_PER_TILE // BLK):
            pltpu.make_async_copy(
                zbuf, acc.at[pl.ds(s * ROWS_PER_TILE + k * BLK, BLK)],
                zsem).wait()
        ih.wait()
        plsc.subcore_barrier()

        @pl.loop(0, WROWS)
        def _(j):
            pltpu.sync_copy(obuf, acc.at[didx.at[j]], add=True)

        @pl.when(w < 4)
        def _():
            pltpu.sync_copy(dst_hbm.at[pl.ds(NW * WROWS + w, 1)],
                            didx.at[pl.ds(WROWS, 1)])
            pltpu.sync_copy(obuf, acc.at[didx.at[WROWS]], add=True)

        plsc.subcore_barrier()

        off = s * ROWS_PER_TILE
        for k in range(4):
            pltpu.sync_copy(
                acc.at[pl.ds(off, ROWS_PER_TILE)],
                out_hbm.at[pl.ds(c * NPAD + off, ROWS_PER_TILE),
                           pl.ds(DEGW * k, DEGW)])

    return degk(dst_r)


def _sc_hop(t, src_r, dst_r):
    @functools.partial(
        pl.kernel,
        mesh=_MESH,
        out_type=jax.ShapeDtypeStruct((2 * NPAD, C), jnp.float32),
        scratch_types=[
            pltpu.VMEM((WROWS + 1, BLK), jnp.int32),
            pltpu.VMEM((WROWS + 1, BLK), jnp.int32),
            pltpu.VMEM((4, BLK, C), jnp.float32),
            pltpu.VMEM_SHARED((NPAD, C), jnp.float32),
            pltpu.SemaphoreType.DMA,
            pltpu.SemaphoreType.DMA,
            pltpu.SemaphoreType.DMA,
            pltpu.SemaphoreType.DMA,
            pltpu.SemaphoreType.DMA,
            pltpu.SemaphoreType.DMA,
            pltpu.SemaphoreType.DMA,
            pltpu.SemaphoreType.DMA,
        ],
        compiler_params=_SC_PARAMS,
    )
    def hop(t_hbm, src_hbm, dst_hbm, out_hbm,
            sidx, didx, bufs, acc, g0, g1, g2, g3, s0, s1, s2, s3):
        c = lax.axis_index("c")
        s = lax.axis_index("s")
        w = c * 16 + s
        gsem = (g0, g1, g2, g3)
        ssem = (s0, s1, s2, s3)

        def gather(blk, b):
            pltpu.async_copy(t_hbm.at[sidx.at[blk]], bufs.at[b], gsem[b])

        def gwait(b):
            pltpu.make_async_copy(t_hbm.at[sidx.at[0]], bufs.at[b],
                                  gsem[b]).wait()

        def scat(blk, b):
            pltpu.async_copy(bufs.at[b], acc.at[didx.at[blk]], ssem[b],
                             add=True)

        def swait(b):
            pltpu.make_async_copy(bufs.at[b], acc.at[didx.at[0]],
                                  ssem[b]).wait()

        ih1 = pltpu.async_copy(src_hbm.at[pl.ds(w * WROWS, WROWS)],
                               sidx.at[pl.ds(0, WROWS)], g2)
        ih2 = pltpu.async_copy(dst_hbm.at[pl.ds(w * WROWS, WROWS)],
                               didx.at[pl.ds(0, WROWS)], g3)

        @pl.loop(0, BLK)
        def _(i):
            for j in range(C // 16):
                bufs[pl.ds(0, 1), pl.ds(i, 1), pl.ds(16 * j, 16)] = jnp.zeros(
                    (1, 1, 16), jnp.float32)

        for k in range(ROWS_PER_TILE // BLK):
            pltpu.async_copy(
                bufs.at[0], acc.at[pl.ds(s * ROWS_PER_TILE + k * BLK, BLK)],
                s0)
        for k in range(ROWS_PER_TILE // BLK):
            pltpu.make_async_copy(
                bufs.at[0], acc.at[pl.ds(s * ROWS_PER_TILE + k * BLK, BLK)],
                s0).wait()
        ih1.wait()
        ih2.wait()

        gather(0, 0)
        gather(1, 1)
        plsc.subcore_barrier()

        gwait(0); scat(0, 0); gather(2, 2)
        gwait(1); scat(1, 1); gather(3, 3)
        gwait(2); scat(2, 2); swait(0); gather(4, 0)
        gwait(3); scat(3, 3); swait(1); gather(5, 1)

        @pl.loop(4, WROWS - 2, step=4)
        def _(j):
            for b in range(4):
                blk = j + b
                gwait(b)
                scat(blk, b)
                nb = (b + 2) % 4
                swait(nb)
                gather(blk + 2, nb)

        gwait(0); scat(WROWS - 2, 0); swait(2)
        gwait(1); scat(WROWS - 1, 1); swait(3)
        swait(0)
        swait(1)

        @pl.when(w < 4)
        def _():
            pltpu.sync_copy(src_hbm.at[pl.ds(NW * WROWS + w, 1)],
                            sidx.at[pl.ds(WROWS, 1)])
            pltpu.sync_copy(dst_hbm.at[pl.ds(NW * WROWS + w, 1)],
                            didx.at[pl.ds(WROWS, 1)])
            pltpu.sync_copy(t_hbm.at[sidx.at[WROWS]], bufs.at[2])
            pltpu.sync_copy(bufs.at[2], acc.at[didx.at[WROWS]], add=True)

        plsc.subcore_barrier()

        off = s * ROWS_PER_TILE
        pltpu.sync_copy(acc.at[pl.ds(off, ROWS_PER_TILE)],
                        out_hbm.at[pl.ds(c * NPAD + off, ROWS_PER_TILE)])

    return hop(t, src_r, dst_r)


_GRID = NH // TCB


def _dinv(dA, dB, i):
    deg = dA + dB + 1.0
    rows = lax.broadcasted_iota(jnp.int32, (TCB, 1), 0) + i * TCB
    return jnp.where(rows < N // 2, lax.rsqrt(deg), 0.0)


def _tc_input(x3, W, degq):
    def body(x_ref, w_ref, da_ref, db_ref, o_ref):
        i = pl.program_id(0)
        s = _dinv(da_ref[...], db_ref[...], i)
        dn = (((1,), (1,)), ((), ()))
        he = lax.dot_general(x_ref[:, 0, :], w_ref[...], dn,
                             preferred_element_type=jnp.float32)
        ho = lax.dot_general(x_ref[:, 1, :], w_ref[...], dn,
                             preferred_element_type=jnp.float32)
        o_ref[...] = jnp.concatenate([he, ho], axis=1) * s

    return pl.pallas_call(
        body,
        grid=(_GRID,),
        in_specs=[
            pl.BlockSpec((TCB, 2, D), lambda i: (i, 0, 0)),
            pl.BlockSpec((C, D), lambda i: (0, 0)),
            pl.BlockSpec((TCB, 128), lambda i: (i, 0)),
            pl.BlockSpec((TCB, 128), lambda i: (_GRID + i, 0)),
        ],
        out_specs=pl.BlockSpec((TCB, 128), lambda i: (i, 0)),
        out_shape=jax.ShapeDtypeStruct((NH, 128), jnp.float32),
    )(x3, W, degq, degq)


def _tc_mid(pv, u2, degq):
    def body(pa_ref, pb_ref, u_ref, da_ref, db_ref, o_ref):
        i = pl.program_id(0)
        s = _dinv(da_ref[...], db_ref[...], i)
        cmb = pa_ref[...] + pb_ref[...] + u_ref[...]
        o_ref[...] = cmb * (s * s)

    return pl.pallas_call(
        body,
        grid=(_GRID,),
        in_specs=[
            pl.BlockSpec((TCB, 128), lambda i: (i, 0)),
            pl.BlockSpec((TCB, 128), lambda i: (_GRID + i, 0)),
            pl.BlockSpec((TCB, 128), lambda i: (i, 0)),
            pl.BlockSpec((TCB, 128), lambda i: (i, 0)),
            pl.BlockSpec((TCB, 128), lambda i: (_GRID + i, 0)),
        ],
        out_specs=pl.BlockSpec((TCB, 128), lambda i: (i, 0)),
        out_shape=jax.ShapeDtypeStruct((NH, 128), jnp.float32),
    )(pv, pv, u2, degq, degq)


def _tc_final(qv, w2, degq, bb):
    def body(qa_ref, qb_ref, w_ref, da_ref, db_ref, b_ref, o_ref):
        i = pl.program_id(0)
        s = _dinv(da_ref[...], db_ref[...], i)
        z = (qa_ref[...] + qb_ref[...] + w_ref[...]) * s + b_ref[...]
        ze = z[:, 0:C]
        zo = z[:, C:128]
        lse_e = jnp.max(ze, axis=1, keepdims=True)
        lse_o = jnp.max(zo, axis=1, keepdims=True)
        lse_e = lse_e + jnp.log(
            jnp.sum(jnp.exp(ze - lse_e), axis=1, keepdims=True))
        lse_o = lse_o + jnp.log(
            jnp.sum(jnp.exp(zo - lse_o), axis=1, keepdims=True))
        o_ref[...] = jnp.concatenate([ze - lse_e, zo - lse_o], axis=1)

    return pl.pallas_call(
        body,
        grid=(_GRID,),
        in_specs=[
            pl.BlockSpec((TCB, 128), lambda i: (i, 0)),
            pl.BlockSpec((TCB, 128), lambda i: (_GRID + i, 0)),
            pl.BlockSpec((TCB, 128), lambda i: (i, 0)),
            pl.BlockSpec((TCB, 128), lambda i: (i, 0)),
            pl.BlockSpec((TCB, 128), lambda i: (_GRID + i, 0)),
            pl.BlockSpec((1, 128), lambda i: (0, 0)),
        ],
        out_specs=pl.BlockSpec((TCB, 128), lambda i: (i, 0)),
        out_shape=jax.ShapeDtypeStruct((NH, 128), jnp.float32),
    )(qv, qv, w2, degq, degq, bb)


def kernel(x, edge_index, W, b):
    src_r = jnp.reshape(edge_index[0].astype(jnp.int32), (EROWS, BLK))
    dst_r = jnp.reshape(edge_index[1].astype(jnp.int32), (EROWS, BLK))
    x3 = jnp.pad(x, ((0, NPAD - N), (0, 0))).reshape(NH, 2, D)
    bb = jnp.reshape(jnp.concatenate([b, b]), (1, 128))

    deg2 = _sc_degree(dst_r)
    degq = jnp.reshape(deg2, (2 * NPAD * 64 // 128, 128))
    u2 = _tc_input(x3, W, degq)
    p = _sc_hop(jnp.reshape(u2, (NPAD, C)), src_r, dst_r)
    w2 = _tc_mid(jnp.reshape(p, (NPAD, 128)), u2, degq)
    q = _sc_hop(jnp.reshape(w2, (NPAD, C)), src_r, dst_r)
    out2 = _tc_final(jnp.reshape(q, (NPAD, 128)), w2, degq, bb)
    return jnp.reshape(out2, (NPAD, C))[0:N]
